# baseline (speedup 1.0000x reference)
import functools

import jax
import jax.numpy as jnp
from jax import lax
from jax.experimental import pallas as pl
from jax.experimental.pallas import tpu as pltpu
from jax.experimental.pallas import tpu_sc as plsc

N_NODES = 10000
N_EDGES = 160000
D_HALF = 128

NC = 2
NS = 16
LANES = 16

EDGES_PER_TILE = N_EDGES // NS
CHUNK = 64
NWIN = 8
WCHUNKS = 20
EDGES_PAD = NWIN * WCHUNKS * CHUNK
ONES_PAD = 64
NBUF = 4
N_ACC = N_NODES + 8
N_DEG = 10016
ROWS_PER_TILE = 624
ROWS_LAST = N_NODES - 15 * ROWS_PER_TILE
DEG_PER_TILE = 1000


def _sc_agg_body(compute_deg, *refs):
    if compute_deg:
        (y_lo, y_hi, src3, dst3, out_lo, out_hi, deg_out,
         src_v, dst_v, r0, r1, r2, r3, ones_v, zdeg_v, acc, deg_acc,
         g0, g1, g2, g3, s0, s1, s2, s3) = refs
    else:
        (y_lo, y_hi, src3, dst3, out_lo, out_hi,
         src_v, dst_v, r0, r1, r2, r3, ones_v, zdeg_v, acc, deg_acc,
         g0, g1, g2, g3, s0, s1, s2, s3) = refs
    rows_v = r0
    bufs = (r0, r1, r2, r3)
    gsems = (g0, g1, g2, g3)
    ssems = (s0, s1, s2, s3)

    c = lax.axis_index("c")
    s = lax.axis_index("s")

    def _zero_rows(i, _):
        for v in range(D_HALF // LANES):
            rows_v[i, pl.ds(v * LANES, LANES)] = jnp.zeros((LANES,), jnp.float32)
        return 0
    lax.fori_loop(0, CHUNK, _zero_rows, 0)

    base_row = s * ROWS_PER_TILE
    nfull = ROWS_PER_TILE // CHUNK
    rem = ROWS_PER_TILE - nfull * CHUNK

    def _zcopy(k, _):
        pltpu.sync_copy(rows_v, acc.at[pl.ds(base_row + k * CHUNK, CHUNK)])
        return 0
    lax.fori_loop(0, nfull, _zcopy, 0)

    @pl.when(s < NS - 1)
    def _():
        pltpu.sync_copy(rows_v.at[pl.ds(0, rem)],
                        acc.at[pl.ds(base_row + nfull * CHUNK, rem)])

    @pl.when(s == NS - 1)
    def _():
        pltpu.sync_copy(rows_v.at[pl.ds(0, ROWS_LAST - nfull * CHUNK)],
                        acc.at[pl.ds(base_row + nfull * CHUNK,
                                     ROWS_LAST - nfull * CHUNK)])

    for v in range(ONES_PAD // LANES):
        ones_v[pl.ds(v * LANES, LANES)] = jnp.ones((LANES,), jnp.float32)
    if compute_deg:
        for v in range(zdeg_v.shape[0] // LANES):
            zdeg_v[pl.ds(v * LANES, LANES)] = jnp.zeros((LANES,), jnp.float32)

        @pl.when(jnp.logical_and(c == 0, s < N_NODES // DEG_PER_TILE))
        def _():
            pltpu.sync_copy(zdeg_v.at[pl.ds(0, DEG_PER_TILE)],
                            deg_acc.at[pl.ds(s * DEG_PER_TILE, DEG_PER_TILE)])

    plsc.subcore_barrier()

    def _edge_loop(y_ref, do_deg):
        def _turn(chunk, b):
            pltpu.make_async_copy(y_ref.at[src_v.at[chunk]], bufs[b],
                                  gsems[b]).wait()
            pltpu.async_copy(bufs[b], acc.at[dst_v.at[chunk]], ssems[b],
                             add=True)
            if do_deg:
                pltpu.sync_copy(ones_v, deg_acc.at[dst_v.at[chunk]], add=True)

            nb = (b + 2) % NBUF

            @pl.when(chunk + 2 < WCHUNKS)
            def _():
                @pl.when(chunk >= 2)
                def _():
                    pltpu.make_async_copy(bufs[nb],
                                          acc.at[dst_v.at[chunk]],
                                          ssems[nb]).wait()
                pltpu.async_copy(y_ref.at[src_v.at[chunk + 2]], bufs[nb],
                                 gsems[nb])

        def window(w, _):
            pltpu.sync_copy(src3.at[s].at[w], src_v)
            pltpu.sync_copy(dst3.at[s].at[w], dst_v)
            pltpu.async_copy(y_ref.at[src_v.at[0]], bufs[0], gsems[0])
            pltpu.async_copy(y_ref.at[src_v.at[1]], bufs[1], gsems[1])

            def body(p, _):
                for b in range(NBUF):
                    _turn(NBUF * p + b, b)
                return 0
            lax.fori_loop(0, WCHUNKS // NBUF, body, 0)
            for t in range(WCHUNKS - 4, WCHUNKS):
                pltpu.make_async_copy(bufs[t % NBUF],
                                      acc.at[dst_v.at[t]],
                                      ssems[t % NBUF]).wait()
            return 0

        lax.fori_loop(0, NWIN, window, 0)

    @pl.when(c == 0)
    def _():
        _edge_loop(y_lo, compute_deg)

    @pl.when(c == 1)
    def _():
        _edge_loop(y_hi, False)

    plsc.subcore_barrier()

    def _copy_out(out_ref):
        @pl.when(s < NS - 1)
        def _():
            pltpu.sync_copy(acc.at[pl.ds(base_row, ROWS_PER_TILE)],
                            out_ref.at[pl.ds(base_row, ROWS_PER_TILE)])

        @pl.when(s == NS - 1)
        def _():
            pltpu.sync_copy(acc.at[pl.ds(base_row, ROWS_LAST)],
                            out_ref.at[pl.ds(base_row, ROWS_LAST)])

    @pl.when(c == 0)
    def _():
        _copy_out(out_lo)

    @pl.when(c == 1)
    def _():
        _copy_out(out_hi)

    if compute_deg:
        @pl.when(jnp.logical_and(c == 0, s < N_NODES // DEG_PER_TILE))
        def _():
            pltpu.sync_copy(deg_acc.at[pl.ds(s * DEG_PER_TILE, DEG_PER_TILE)],
                            zdeg_v.at[pl.ds(0, DEG_PER_TILE)])
            pltpu.sync_copy(zdeg_v.at[pl.ds(0, DEG_PER_TILE)],
                            deg_out.at[pl.ds(s * DEG_PER_TILE, DEG_PER_TILE)])


def _make_sc_agg(compute_deg):
    out_type = [
        jax.ShapeDtypeStruct((N_NODES, D_HALF), jnp.float32),
        jax.ShapeDtypeStruct((N_NODES, D_HALF), jnp.float32),
    ]
    if compute_deg:
        out_type.append(jax.ShapeDtypeStruct((N_NODES,), jnp.float32))
    scratch = (
        [
            pltpu.VMEM((WCHUNKS, CHUNK), jnp.int32),
            pltpu.VMEM((WCHUNKS, CHUNK), jnp.int32),
        ]
        + [pltpu.VMEM((CHUNK, D_HALF), jnp.float32)] * NBUF
        + [
            pltpu.VMEM((ONES_PAD,), jnp.float32),
            pltpu.VMEM((1024,), jnp.float32),
            pltpu.VMEM_SHARED((N_ACC, D_HALF), jnp.float32),
            pltpu.VMEM_SHARED((N_DEG,), jnp.float32),
        ]
        + [pltpu.SemaphoreType.DMA] * (2 * NBUF)
    )
    mesh = plsc.VectorSubcoreMesh(core_axis_name="c", subcore_axis_name="s",
                                  num_cores=NC, num_subcores=NS)
    return pl.kernel(
        functools.partial(_sc_agg_body, compute_deg),
        out_type=out_type, mesh=mesh, scratch_types=scratch,
        name="sc_agg_deg" if compute_deg else "sc_agg",
    )


_BM = 2000


def _mm1_body(x_ref, w_ref, lo_ref, hi_ref):
    y = jnp.dot(x_ref[...], w_ref[...], preferred_element_type=jnp.float32)
    lo_ref[...] = y[:, :D_HALF]
    hi_ref[...] = y[:, D_HALF:]


def _tc_mm1(x, W1):
    n, d = x.shape
    grid = n // _BM
    return pl.pallas_call(
        _mm1_body,
        grid=(grid,),
        in_specs=[
            pl.BlockSpec((_BM, d), lambda i: (i, 0)),
            pl.BlockSpec((d, W1.shape[1]), lambda i: (0, 0)),
        ],
        out_specs=[
            pl.BlockSpec((_BM, D_HALF), lambda i: (i, 0)),
            pl.BlockSpec((_BM, D_HALF), lambda i: (i, 0)),
        ],
        out_shape=[
            jax.ShapeDtypeStruct((n, D_HALF), jnp.float32),
            jax.ShapeDtypeStruct((n, D_HALF), jnp.float32),
        ],
    )(x, W1)


def _mid_body(slo_ref, shi_ref, deg_ref, b1_ref, w2_ref, lo_ref, hi_ref):
    inv = 1.0 / jnp.maximum(deg_ref[...], 1.0)
    h_lo = jnp.maximum(slo_ref[...] * inv + b1_ref[:, :D_HALF], 0.0)
    h_hi = jnp.maximum(shi_ref[...] * inv + b1_ref[:, D_HALF:], 0.0)
    y = (jnp.dot(h_lo, w2_ref[:D_HALF, :], preferred_element_type=jnp.float32)
         + jnp.dot(h_hi, w2_ref[D_HALF:, :], preferred_element_type=jnp.float32))
    lo_ref[...] = y[:, :D_HALF]
    hi_ref[...] = y[:, D_HALF:]


def _tc_mid(s_lo, s_hi, deg, b1, W2):
    n = s_lo.shape[0]
    grid = n // _BM
    return pl.pallas_call(
        _mid_body,
        grid=(grid,),
        in_specs=[
            pl.BlockSpec((_BM, D_HALF), lambda i: (i, 0)),
            pl.BlockSpec((_BM, D_HALF), lambda i: (i, 0)),
            pl.BlockSpec((_BM, 1), lambda i: (i, 0)),
            pl.BlockSpec((1, 2 * D_HALF), lambda i: (0, 0)),
            pl.BlockSpec(W2.shape, lambda i: (0, 0)),
        ],
        out_specs=[
            pl.BlockSpec((_BM, D_HALF), lambda i: (i, 0)),
            pl.BlockSpec((_BM, D_HALF), lambda i: (i, 0)),
        ],
        out_shape=[
            jax.ShapeDtypeStruct((n, D_HALF), jnp.float32),
            jax.ShapeDtypeStruct((n, D_HALF), jnp.float32),
        ],
    )(s_lo, s_hi, deg, b1, W2)


def _final_body(tlo_ref, thi_ref, deg_ref, b2_ref, out_ref):
    inv = 1.0 / jnp.maximum(deg_ref[...], 1.0)
    out_ref[:, :D_HALF] = tlo_ref[...] * inv + b2_ref[:, :D_HALF]
    out_ref[:, D_HALF:] = thi_ref[...] * inv + b2_ref[:, D_HALF:]


def _tc_final(t_lo, t_hi, deg, b2):
    n = t_lo.shape[0]
    grid = n // _BM
    return pl.pallas_call(
        _final_body,
        grid=(grid,),
        in_specs=[
            pl.BlockSpec((_BM, D_HALF), lambda i: (i, 0)),
            pl.BlockSpec((_BM, D_HALF), lambda i: (i, 0)),
            pl.BlockSpec((_BM, 1), lambda i: (i, 0)),
            pl.BlockSpec((1, 2 * D_HALF), lambda i: (0, 0)),
        ],
        out_specs=pl.BlockSpec((_BM, 2 * D_HALF), lambda i: (i, 0)),
        out_shape=jax.ShapeDtypeStruct((n, 2 * D_HALF), jnp.float32),
    )(t_lo, t_hi, deg, b2)


def kernel(x, edge_index, W1, b1, W2, b2):
    src = edge_index[0].astype(jnp.int32)
    dst = edge_index[1].astype(jnp.int32)
    npad = EDGES_PAD - EDGES_PER_TILE
    src3 = jnp.concatenate(
        [src.reshape(NS, EDGES_PER_TILE),
         jnp.zeros((NS, npad), jnp.int32)], axis=1
    ).reshape(NS, NWIN, WCHUNKS, CHUNK)
    dst3 = jnp.concatenate(
        [dst.reshape(NS, EDGES_PER_TILE),
         jnp.full((NS, npad), N_NODES, jnp.int32)], axis=1
    ).reshape(NS, NWIN, WCHUNKS, CHUNK)
    b1r = b1.reshape(1, -1)
    b2r = b2.reshape(1, -1)

    y_lo, y_hi = _tc_mm1(x, W1)
    s_lo, s_hi, deg = _make_sc_agg(True)(y_lo, y_hi, src3, dst3)
    degc = deg.reshape(N_NODES, 1)
    y2_lo, y2_hi = _tc_mid(s_lo, s_hi, degc, b1r, W2)
    t_lo, t_hi = _make_sc_agg(False)(y2_lo, y2_hi, src3, dst3)
    return _tc_final(t_lo, t_hi, degc, b2r)


# speedup vs baseline: 2.2462x; 2.2462x over previous
import functools

import jax
import jax.numpy as jnp
from jax import lax
from jax.experimental import pallas as pl
from jax.experimental.pallas import tpu as pltpu
from jax.experimental.pallas import tpu_sc as plsc

N_NODES = 10000
N_EDGES = 160000
D_HALF = 128

NC = 2
NS = 16
LANES = 16

EDGES_PER_TILE = N_EDGES // NS
CHUNK = 80
NWIN = 5
WCHUNKS = 25
ONES_PAD = 80
NBUF = 3
ROWS_PER_TILE = 624
ROWS_LAST = N_NODES - 15 * ROWS_PER_TILE
DEG_PER_TILE = 1000


def _sc_agg_body(compute_deg, *refs):
    if compute_deg:
        (y_lo, y_hi, src3, dst3, out_lo, out_hi, deg_out,
         src_v, dst_v, rows_v, rows_b, rows_c, ones_v, zdeg_v, acc, deg_acc,
         sem, sem_b, sem_c) = refs
    else:
        (y_lo, y_hi, src3, dst3, out_lo, out_hi,
         src_v, dst_v, rows_v, rows_b, rows_c, ones_v, zdeg_v, acc, deg_acc,
         sem, sem_b, sem_c) = refs

    c = lax.axis_index("c")
    s = lax.axis_index("s")
    bufs = (rows_v, rows_b, rows_c)[:NBUF]
    sems = (sem, sem_b, sem_c)[:NBUF]

    pltpu.sync_copy(src3.at[s].at[0], src_v)
    pltpu.sync_copy(dst3.at[s].at[0], dst_v)

    def _prime2(y_ref):
        for b in range(NBUF - 1):
            pltpu.async_copy(y_ref.at[src_v.at[b]], bufs[b], sems[b])

    @pl.when(c == 0)
    def _():
        _prime2(y_lo)

    @pl.when(c == 1)
    def _():
        _prime2(y_hi)

    zbuf = bufs[NBUF - 1]

    def _zero_rows(i, _):
        for v in range(D_HALF // LANES):
            zbuf[i, pl.ds(v * LANES, LANES)] = jnp.zeros((LANES,), jnp.float32)
        return 0
    lax.fori_loop(0, CHUNK, _zero_rows, 0)

    base_row = s * ROWS_PER_TILE
    nfull = ROWS_PER_TILE // CHUNK
    rem = ROWS_PER_TILE - nfull * CHUNK

    def _zcopy(k, _):
        pltpu.sync_copy(zbuf, acc.at[pl.ds(base_row + k * CHUNK, CHUNK)])
        return 0
    lax.fori_loop(0, nfull, _zcopy, 0)

    @pl.when(s < NS - 1)
    def _():
        pltpu.sync_copy(zbuf.at[pl.ds(0, rem)],
                        acc.at[pl.ds(base_row + nfull * CHUNK, rem)])

    @pl.when(s == NS - 1)
    def _():
        pltpu.sync_copy(zbuf.at[pl.ds(0, ROWS_LAST - nfull * CHUNK)],
                        acc.at[pl.ds(base_row + nfull * CHUNK,
                                     ROWS_LAST - nfull * CHUNK)])

    @pl.when(c == 0)
    def _():
        pltpu.async_copy(y_lo.at[src_v.at[NBUF - 1]], zbuf, sems[NBUF - 1])

    @pl.when(c == 1)
    def _():
        pltpu.async_copy(y_hi.at[src_v.at[NBUF - 1]], zbuf, sems[NBUF - 1])

    for v in range(ONES_PAD // LANES):
        ones_v[pl.ds(v * LANES, LANES)] = jnp.ones((LANES,), jnp.float32)
    if compute_deg:
        for v in range(zdeg_v.shape[0] // LANES):
            zdeg_v[pl.ds(v * LANES, LANES)] = jnp.zeros((LANES,), jnp.float32)

        @pl.when(jnp.logical_and(c == 0, s < N_NODES // DEG_PER_TILE))
        def _():
            pltpu.sync_copy(zdeg_v.at[pl.ds(0, DEG_PER_TILE)],
                            deg_acc.at[pl.ds(s * DEG_PER_TILE, DEG_PER_TILE)])

    plsc.subcore_barrier()

    def _edge_loop(y_ref, do_deg):
        def _step(chunk, buf, sm):
            pltpu.make_async_copy(y_ref.at[src_v.at[chunk]], buf, sm).wait()
            pltpu.sync_copy(buf, acc.at[dst_v.at[chunk]], add=True)
            if do_deg:
                pltpu.sync_copy(ones_v.at[pl.ds(0, CHUNK)],
                                deg_acc.at[dst_v.at[chunk]], add=True)

            @pl.when(chunk + NBUF < WCHUNKS)
            def _():
                pltpu.async_copy(y_ref.at[src_v.at[chunk + NBUF]], buf, sm)

        def window(w, _):
            @pl.when(w > 0)
            def _():
                pltpu.sync_copy(src3.at[s].at[w], src_v)
                pltpu.sync_copy(dst3.at[s].at[w], dst_v)
                for b in range(NBUF):
                    pltpu.async_copy(y_ref.at[src_v.at[b]], bufs[b], sems[b])

            def body(p, _):
                for b in range(NBUF):
                    _step(NBUF * p + b, bufs[b], sems[b])
                return 0
            lax.fori_loop(0, WCHUNKS // NBUF, body, 0)
            for t in range(WCHUNKS - (WCHUNKS // NBUF) * NBUF):
                _step((WCHUNKS // NBUF) * NBUF + t, bufs[t], sems[t])
            return 0

        lax.fori_loop(0, NWIN, window, 0)

    @pl.when(c == 0)
    def _():
        _edge_loop(y_lo, compute_deg)

    @pl.when(c == 1)
    def _():
        _edge_loop(y_hi, False)

    plsc.subcore_barrier()

    def _copy_out(out_ref):
        @pl.when(s < NS - 1)
        def _():
            pltpu.sync_copy(acc.at[pl.ds(base_row, ROWS_PER_TILE)],
                            out_ref.at[pl.ds(base_row, ROWS_PER_TILE)])

        @pl.when(s == NS - 1)
        def _():
            pltpu.sync_copy(acc.at[pl.ds(base_row, ROWS_LAST)],
                            out_ref.at[pl.ds(base_row, ROWS_LAST)])

    @pl.when(c == 0)
    def _():
        _copy_out(out_lo)

    @pl.when(c == 1)
    def _():
        _copy_out(out_hi)

    if compute_deg:
        @pl.when(jnp.logical_and(c == 0, s < N_NODES // DEG_PER_TILE))
        def _():
            pltpu.sync_copy(deg_acc.at[pl.ds(s * DEG_PER_TILE, DEG_PER_TILE)],
                            zdeg_v.at[pl.ds(0, DEG_PER_TILE)])
            pltpu.sync_copy(zdeg_v.at[pl.ds(0, DEG_PER_TILE)],
                            deg_out.at[pl.ds(s * DEG_PER_TILE, DEG_PER_TILE)])


def _make_sc_agg(compute_deg):
    out_type = [
        jax.ShapeDtypeStruct((N_NODES, D_HALF), jnp.float32),
        jax.ShapeDtypeStruct((N_NODES, D_HALF), jnp.float32),
    ]
    if compute_deg:
        out_type.append(jax.ShapeDtypeStruct((N_NODES,), jnp.float32))
    scratch = [
        pltpu.VMEM((WCHUNKS, CHUNK), jnp.int32),
        pltpu.VMEM((WCHUNKS, CHUNK), jnp.int32),
        pltpu.VMEM((CHUNK, D_HALF), jnp.float32),
        pltpu.VMEM((CHUNK, D_HALF), jnp.float32),
        pltpu.VMEM((CHUNK, D_HALF), jnp.float32),
        pltpu.VMEM((ONES_PAD,), jnp.float32),
        pltpu.VMEM((1024,), jnp.float32),
        pltpu.VMEM_SHARED((N_NODES, D_HALF), jnp.float32),
        pltpu.VMEM_SHARED((N_NODES,), jnp.float32),
        pltpu.SemaphoreType.DMA,
        pltpu.SemaphoreType.DMA,
        pltpu.SemaphoreType.DMA,
    ]
    mesh = plsc.VectorSubcoreMesh(core_axis_name="c", subcore_axis_name="s",
                                  num_cores=NC, num_subcores=NS)
    return pl.kernel(
        functools.partial(_sc_agg_body, compute_deg),
        out_type=out_type, mesh=mesh, scratch_types=scratch,
        name="sc_agg_deg" if compute_deg else "sc_agg",
    )


_BM = 2000


def _mm1_body(x_ref, w_ref, lo_ref, hi_ref):
    y = jnp.dot(x_ref[...], w_ref[...], preferred_element_type=jnp.float32)
    lo_ref[...] = y[:, :D_HALF]
    hi_ref[...] = y[:, D_HALF:]


def _tc_mm1(x, W1):
    n, d = x.shape
    grid = n // _BM
    return pl.pallas_call(
        _mm1_body,
        grid=(grid,),
        in_specs=[
            pl.BlockSpec((_BM, d), lambda i: (i, 0)),
            pl.BlockSpec((d, W1.shape[1]), lambda i: (0, 0)),
        ],
        out_specs=[
            pl.BlockSpec((_BM, D_HALF), lambda i: (i, 0)),
            pl.BlockSpec((_BM, D_HALF), lambda i: (i, 0)),
        ],
        out_shape=[
            jax.ShapeDtypeStruct((n, D_HALF), jnp.float32),
            jax.ShapeDtypeStruct((n, D_HALF), jnp.float32),
        ],
    )(x, W1)


def _mid_body(slo_ref, shi_ref, deg_ref, b1_ref, w2_ref, lo_ref, hi_ref):
    inv = 1.0 / jnp.maximum(deg_ref[...], 1.0)
    h_lo = jnp.maximum(slo_ref[...] * inv + b1_ref[:, :D_HALF], 0.0)
    h_hi = jnp.maximum(shi_ref[...] * inv + b1_ref[:, D_HALF:], 0.0)
    y = (jnp.dot(h_lo, w2_ref[:D_HALF, :], preferred_element_type=jnp.float32)
         + jnp.dot(h_hi, w2_ref[D_HALF:, :], preferred_element_type=jnp.float32))
    lo_ref[...] = y[:, :D_HALF]
    hi_ref[...] = y[:, D_HALF:]


def _tc_mid(s_lo, s_hi, deg, b1, W2):
    n = s_lo.shape[0]
    grid = n // _BM
    return pl.pallas_call(
        _mid_body,
        grid=(grid,),
        in_specs=[
            pl.BlockSpec((_BM, D_HALF), lambda i: (i, 0)),
            pl.BlockSpec((_BM, D_HALF), lambda i: (i, 0)),
            pl.BlockSpec((_BM, 1), lambda i: (i, 0)),
            pl.BlockSpec((1, 2 * D_HALF), lambda i: (0, 0)),
            pl.BlockSpec(W2.shape, lambda i: (0, 0)),
        ],
        out_specs=[
            pl.BlockSpec((_BM, D_HALF), lambda i: (i, 0)),
            pl.BlockSpec((_BM, D_HALF), lambda i: (i, 0)),
        ],
        out_shape=[
            jax.ShapeDtypeStruct((n, D_HALF), jnp.float32),
            jax.ShapeDtypeStruct((n, D_HALF), jnp.float32),
        ],
    )(s_lo, s_hi, deg, b1, W2)


def _final_body(tlo_ref, thi_ref, deg_ref, b2_ref, out_ref):
    inv = 1.0 / jnp.maximum(deg_ref[...], 1.0)
    out_ref[:, :D_HALF] = tlo_ref[...] * inv + b2_ref[:, :D_HALF]
    out_ref[:, D_HALF:] = thi_ref[...] * inv + b2_ref[:, D_HALF:]


def _tc_final(t_lo, t_hi, deg, b2):
    n = t_lo.shape[0]
    grid = n // _BM
    return pl.pallas_call(
        _final_body,
        grid=(grid,),
        in_specs=[
            pl.BlockSpec((_BM, D_HALF), lambda i: (i, 0)),
            pl.BlockSpec((_BM, D_HALF), lambda i: (i, 0)),
            pl.BlockSpec((_BM, 1), lambda i: (i, 0)),
            pl.BlockSpec((1, 2 * D_HALF), lambda i: (0, 0)),
        ],
        out_specs=pl.BlockSpec((_BM, 2 * D_HALF), lambda i: (i, 0)),
        out_shape=jax.ShapeDtypeStruct((n, 2 * D_HALF), jnp.float32),
    )(t_lo, t_hi, deg, b2)


def kernel(x, edge_index, W1, b1, W2, b2):
    src = edge_index[0].astype(jnp.int32)
    dst = edge_index[1].astype(jnp.int32)
    src3 = src.reshape(NS, NWIN, WCHUNKS, CHUNK)
    dst3 = dst.reshape(NS, NWIN, WCHUNKS, CHUNK)
    b1r = b1.reshape(1, -1)
    b2r = b2.reshape(1, -1)

    y_lo, y_hi = _tc_mm1(x, W1)
    s_lo, s_hi, deg = _make_sc_agg(True)(y_lo, y_hi, src3, dst3)
    degc = deg.reshape(N_NODES, 1)
    y2_lo, y2_hi = _tc_mid(s_lo, s_hi, degc, b1r, W2)
    t_lo, t_hi = _make_sc_agg(False)(y2_lo, y2_hi, src3, dst3)
    return _tc_final(t_lo, t_hi, degc, b2r)
